# baseline (speedup 1.0000x reference)
import functools

import jax
import jax.numpy as jnp
from jax import lax
from jax.experimental import pallas as pl
from jax.experimental.pallas import tpu as pltpu
from jax.experimental.pallas import tpu_sc as plsc

L = 200
B = 4096
D = 32
DP = 128
CHUNK = 128
NC, NS = 2, 16
NW = NC * NS
NCHUNKS = (L * B) // CHUNK
CPW = NCHUNKS // NW
CPL = B // CHUNK
NBUF = 5
PG = 2
KS = 2

_mesh = plsc.VectorSubcoreMesh(
    core_axis_name="c", subcore_axis_name="s", num_cores=NC, num_subcores=NS
)


@functools.partial(
    pl.kernel,
    out_type=jax.ShapeDtypeStruct((L, B, D), jnp.float32),
    mesh=_mesh,
    scratch_types=[
        pltpu.VMEM((CPW, CHUNK), jnp.int32),
        pltpu.VMEM((NBUF, CHUNK, D), jnp.float32),
        pltpu.SemaphoreType.DMA,
        pltpu.SemaphoreType.DMA,
    ],
    compiler_params=pltpu.CompilerParams(use_tc_tiling_on_sc=False),
)
def _embed_sc(table_hbm, idx_hbm, out_hbm, idx_v, ring, sem_g, sem_s):
    wid = lax.axis_index("s") * NC + lax.axis_index("c")
    c0 = wid * CPW

    pltpu.sync_copy(idx_hbm.at[pl.ds(c0, CPW)], idx_v)

    def issue_gather(j):
        pltpu.async_copy(table_hbm.at[idx_v.at[j]], ring.at[j % NBUF], sem_g)

    def wait_gather():
        pltpu.make_async_copy(
            table_hbm.at[idx_v.at[0]], ring.at[0], sem_g
        ).wait()

    def issue_scatter(j):
        c = c0 + j
        pltpu.async_copy(
            ring.at[j % NBUF],
            out_hbm.at[c // CPL, pl.ds((c % CPL) * CHUNK, CHUNK)],
            sem_s,
        )

    def wait_scatter():
        pltpu.make_async_copy(
            ring.at[0], out_hbm.at[0, pl.ds(0, CHUNK)], sem_s
        ).wait()

    for j in range(PG + 1):
        issue_gather(j)

    def body(j, _):
        wait_gather()
        issue_scatter(j)

        @pl.when(j >= KS)
        def _():
            wait_scatter()

        @pl.when(j + PG + 1 < CPW)
        def _():
            issue_gather(j + PG + 1)

        return 0

    lax.fori_loop(0, CPW, body, 0)
    for _ in range(KS):
        wait_scatter()


def kernel(x, table):
    idx = jnp.transpose(x).reshape(NCHUNKS, CHUNK)
    return _embed_sc(table, idx)


# speedup vs baseline: 1.5843x; 1.5843x over previous
import functools

import jax
import jax.numpy as jnp
from jax import lax
from jax.experimental import pallas as pl
from jax.experimental.pallas import tpu as pltpu
from jax.experimental.pallas import tpu_sc as plsc

L = 200
B = 4096
D = 32
DP = 128
CHUNK = 128
NC, NS = 2, 16
NW = NC * NS
NCHUNKS = (L * B) // CHUNK
CPW = NCHUNKS // NW
CPL = B // CHUNK
NBUF = 5
PG = 2
KS = 2

_mesh = plsc.VectorSubcoreMesh(
    core_axis_name="c", subcore_axis_name="s", num_cores=NC, num_subcores=NS
)


@functools.partial(
    pl.kernel,
    out_type=jax.ShapeDtypeStruct((NCHUNKS, CHUNK, D), jnp.float32),
    mesh=_mesh,
    scratch_types=[
        pltpu.VMEM((CPW, CHUNK), jnp.int32),
        pltpu.VMEM((NBUF, CHUNK, D), jnp.float32),
        pltpu.SemaphoreType.DMA,
        pltpu.SemaphoreType.DMA,
    ],
    compiler_params=pltpu.CompilerParams(use_tc_tiling_on_sc=False),
)
def _embed_sc(table_hbm, idx_hbm, out_hbm, idx_v, ring, sem_g, sem_s):
    wid = lax.axis_index("s") * NC + lax.axis_index("c")
    c0 = wid * CPW

    pltpu.sync_copy(idx_hbm.at[pl.ds(c0, CPW)], idx_v)

    def issue_gather(j):
        pltpu.async_copy(table_hbm.at[idx_v.at[j]], ring.at[j % NBUF], sem_g)

    def wait_gather():
        pltpu.make_async_copy(
            table_hbm.at[idx_v.at[0]], ring.at[0], sem_g
        ).wait()

    def issue_scatter(j):
        pltpu.async_copy(ring.at[j % NBUF], out_hbm.at[c0 + j], sem_s)

    def wait_scatter():
        pltpu.make_async_copy(ring.at[0], out_hbm.at[0], sem_s).wait()

    for j in range(PG + 1):
        issue_gather(j)

    def body(j, _):
        wait_gather()
        issue_scatter(j)

        @pl.when(j >= KS)
        def _():
            wait_scatter()

        @pl.when(j + PG + 1 < CPW)
        def _():
            issue_gather(j + PG + 1)

        return 0

    lax.fori_loop(0, CPW, body, 0)
    for _ in range(KS):
        wait_scatter()


ROWS_PER_L = (B * D) // 128


def _transpose_block(flat_ref, out_ref):
    xt = flat_ref[...].T
    y = xt.reshape(4, D, ROWS_PER_L).transpose(1, 0, 2).reshape(D, B)
    out_ref[0] = y


_to_ldb = pl.pallas_call(
    _transpose_block,
    grid=(L,),
    in_specs=[pl.BlockSpec((ROWS_PER_L, 128), lambda l: (l, 0))],
    out_specs=pl.BlockSpec((1, D, B), lambda l: (l, 0, 0)),
    out_shape=jax.ShapeDtypeStruct((L, D, B), jnp.float32),
)


def kernel(x, table):
    xt = jnp.transpose(x)
    idx = (
        xt.reshape(L, 4, 32, 32)
        .transpose(0, 2, 3, 1)
        .reshape(NCHUNKS, CHUNK)
    )
    gathered = _embed_sc(table, idx)
    flat = gathered.reshape(L * ROWS_PER_L, 128)
    ldb = _to_ldb(flat)
    return jnp.transpose(ldb, (0, 2, 1))
